# baseline (speedup 1.0000x reference)
import jax
import jax.numpy as jnp
from jax import lax
from jax.experimental import pallas as pl
from jax.experimental.pallas import tpu as pltpu
from jax.experimental.pallas import tpu_sc as plsc

C = 96
H = 128
W = 128
SIDE = 14
P = SIDE * SIDE
NB = 1000
NBP = 1024
NW = 32
BPW = NBP // NW
PW = 17
PATCH_WORDS = PW * PW * C
OBLK = C * P
ROW_WORDS = PW * C


def _tap_data(coord_lo, coord_hi, limit):
    jf = lax.iota(jnp.int32, 16).astype(jnp.float32)
    pc = (jf + 0.5) / float(SIDE)
    p_img = pc * (coord_hi - coord_lo) + coord_lo
    ix = (p_img * 0.5 - 1.0) * 0.5
    t = ix.astype(jnp.int32)
    tf = t.astype(jnp.float32)
    ione = jnp.full((16,), 1, jnp.int32)
    izero = jnp.full((16,), 0, jnp.int32)
    fone = jnp.full((16,), 1.0, jnp.float32)
    fzero = jnp.full((16,), 0.0, jnp.float32)
    i0 = t - jnp.where(tf > ix, ione, izero)
    i0f = i0.astype(jnp.float32)
    w1 = ix - i0f
    w0 = 1.0 - w1
    w0 = w0 * jnp.where(i0 >= 0, fone, fzero)
    w1 = w1 * jnp.where(i0 <= limit - 2, fone, fzero)
    return i0, i0 + 1, w0, w1


def _scalar_base(lo, hi, limit):
    pc0 = (0.0 + 0.5) / float(SIDE)
    p_img = pc0 * (hi - lo) + lo
    ix = (p_img * 0.5 - 1.0) * 0.5
    return jnp.clip(ix.astype(jnp.int32), 0, limit - PW)


def _body(table_hbm, boxes_hbm, out_hbm,
          boxes_v, jxi0_v, jxw0_v, idat0_v, jxi1_v, jxw1_v, idat1_v,
          patch0_v, patch1_v, ob0_v, ob1_v,
          ps0, ps1, os0, os1):
    cid = lax.axis_index("c")
    sid = lax.axis_index("s")
    wid = sid * 2 + cid
    r0 = wid * BPW

    pltpu.sync_copy(boxes_hbm.at[pl.ds(r0 * 4, BPW * 4)],
                    boxes_v.at[pl.ds(0, BPW * 4)])

    ch_iota = lax.iota(jnp.int32, 16)
    strides = [ch_iota * P + k * 16 * P for k in range(6)]
    idx4 = ch_iota * 4
    mask14 = ch_iota < SIDE

    def fire(b, patch_ref, psem, jxi, jxw, idt):
        @pl.when(b < BPW)
        def _():
            bv = boxes_v[pl.ds(b * 4, 16)]
            xi0, xi1, wx0, wx1 = _tap_data(bv[0], bv[2], W)
            yi0, yi1, wy0, wy1 = _tap_data(bv[1], bv[3], H)
            bx = jnp.clip(xi0[0], 0, W - PW)
            by = jnp.clip(yi0[0], 0, H - PW)
            jxi[pl.ds(0, 16)] = jnp.clip(xi0 - bx, 0, PW - 1) * C
            jxi[pl.ds(16, 16)] = jnp.clip(xi1 - bx, 0, PW - 1) * C
            jxw[pl.ds(0, 16)] = wx0
            jxw[pl.ds(16, 16)] = wx1
            rf = jnp.full((16,), float(ROW_WORDS), jnp.float32)
            plsc.store_scatter(
                idt, [idx4 + 0],
                jnp.clip(yi0 - by, 0, PW - 1).astype(jnp.float32) * rf)
            plsc.store_scatter(
                idt, [idx4 + 1],
                jnp.clip(yi1 - by, 0, PW - 1).astype(jnp.float32) * rf)
            plsc.store_scatter(idt, [idx4 + 2], wy0)
            plsc.store_scatter(idt, [idx4 + 3], wy1)
            base_row = (by * W + bx) * C
            for k in range(PW):
                pltpu.async_copy(
                    table_hbm.at[pl.ds(base_row + k * (W * C), ROW_WORDS)],
                    patch_ref.at[pl.ds(k * ROW_WORDS, ROW_WORDS)],
                    psem,
                )

    def work(b, patch_ref, psem, jxi, jxw, idt, ob_ref, osem):
        pltpu.make_async_copy(
            table_hbm.at[pl.ds(0, PATCH_WORDS)], patch_ref, psem).wait()

        @pl.when(b >= 2)
        def _():
            pltpu.make_async_copy(
                out_hbm.at[pl.ds(0, OBLK)], ob_ref, osem).wait()

        c0v = jxi[pl.ds(0, 16)]
        c1v = jxi[pl.ds(16, 16)]
        wx0v = jxw[pl.ds(0, 16)]
        wx1v = jxw[pl.ds(16, 16)]

        @plsc.parallel_loop(0, SIDE, unroll=1)
        def i_body(i):
            idv = idt[pl.ds(i * 4, 16)]
            row0 = idv[0].astype(jnp.int32)
            row1 = idv[1].astype(jnp.int32)
            wy0i = idv[2]
            wy1i = idv[3]
            a00 = c0v + row0
            a01 = c1v + row0
            a10 = c0v + row1
            a11 = c1v + row1
            w00 = wx0v * wy0i
            w01 = wx1v * wy0i
            w10 = wx0v * wy1i
            w11 = wx1v * wy1i
            obase = ch_iota + i * SIDE
            for c in range(C):
                v = (plsc.load_gather(patch_ref, [a00 + c]) * w00
                     + plsc.load_gather(patch_ref, [a01 + c]) * w01
                     + plsc.load_gather(patch_ref, [a10 + c]) * w10
                     + plsc.load_gather(patch_ref, [a11 + c]) * w11)
                plsc.store_scatter(ob_ref, [obase + c * P], v, mask=mask14)

        pltpu.async_copy(ob_ref, out_hbm.at[pl.ds((r0 + b) * OBLK, OBLK)],
                         osem)

    fire(0, patch0_v, ps0, jxi0_v, jxw0_v, idat0_v)
    fire(1, patch1_v, ps1, jxi1_v, jxw1_v, idat1_v)

    def g_body(g, _):
        b0 = g * 2
        work(b0, patch0_v, ps0, jxi0_v, jxw0_v, idat0_v, ob0_v, os0)
        fire(b0 + 2, patch0_v, ps0, jxi0_v, jxw0_v, idat0_v)
        work(b0 + 1, patch1_v, ps1, jxi1_v, jxw1_v, idat1_v, ob1_v, os1)
        fire(b0 + 3, patch1_v, ps1, jxi1_v, jxw1_v, idat1_v)
        return ()

    lax.fori_loop(0, BPW // 2, g_body, (), unroll=False)

    pltpu.make_async_copy(out_hbm.at[pl.ds(0, OBLK)], ob0_v, os0).wait()
    pltpu.make_async_copy(out_hbm.at[pl.ds(0, OBLK)], ob1_v, os1).wait()


@jax.jit
def kernel(feature0, pred_boxes):
    table = jnp.transpose(feature0[0].reshape(C, H * W)).reshape(-1)
    boxes = jnp.pad(pred_boxes, ((0, NBP - NB), (0, 0))).reshape(-1)

    run = pl.kernel(
        _body,
        out_type=jax.ShapeDtypeStruct((NBP * OBLK,), jnp.float32),
        mesh=plsc.VectorSubcoreMesh(core_axis_name="c", subcore_axis_name="s"),
        compiler_params=pltpu.CompilerParams(needs_layout_passes=False),
        scratch_types=[
            pltpu.VMEM((BPW * 4 + 24,), jnp.float32),
            pltpu.VMEM((32,), jnp.int32),
            pltpu.VMEM((32,), jnp.float32),
            pltpu.VMEM((80,), jnp.float32),
            pltpu.VMEM((32,), jnp.int32),
            pltpu.VMEM((32,), jnp.float32),
            pltpu.VMEM((80,), jnp.float32),
            pltpu.VMEM((PATCH_WORDS,), jnp.float32),
            pltpu.VMEM((PATCH_WORDS,), jnp.float32),
            pltpu.VMEM((OBLK,), jnp.float32),
            pltpu.VMEM((OBLK,), jnp.float32),
            pltpu.SemaphoreType.DMA,
            pltpu.SemaphoreType.DMA,
            pltpu.SemaphoreType.DMA,
            pltpu.SemaphoreType.DMA,
        ],
    )
    out = run(table, boxes)
    return out.reshape(NBP, C, P)[:NB]


# speedup vs baseline: 3.3246x; 3.3246x over previous
import jax
import jax.numpy as jnp
from jax import lax
from jax.experimental import pallas as pl
from jax.experimental.pallas import tpu as pltpu
from jax.experimental.pallas import tpu_sc as plsc

C = 96
H = 128
W = 128
SIDE = 14
P = SIDE * SIDE
NB = 1000
NBP = 1024
NW = 32
BPW = NBP // NW
PW = 17
PATCH_WORDS = PW * PW * C
OBLK = C * P
ROW_WORDS = PW * C


def _tap_data(coord_lo, coord_hi, limit):
    jf = lax.iota(jnp.int32, 16).astype(jnp.float32)
    pc = (jf + 0.5) / float(SIDE)
    p_img = pc * (coord_hi - coord_lo) + coord_lo
    ix = (p_img * 0.5 - 1.0) * 0.5
    t = ix.astype(jnp.int32)
    tf = t.astype(jnp.float32)
    ione = jnp.full((16,), 1, jnp.int32)
    izero = jnp.full((16,), 0, jnp.int32)
    fone = jnp.full((16,), 1.0, jnp.float32)
    fzero = jnp.full((16,), 0.0, jnp.float32)
    i0 = t - jnp.where(tf > ix, ione, izero)
    i0f = i0.astype(jnp.float32)
    w1 = ix - i0f
    w0 = 1.0 - w1
    w0 = w0 * jnp.where(i0 >= 0, fone, fzero)
    w1 = w1 * jnp.where(i0 <= limit - 2, fone, fzero)
    return i0, i0 + 1, w0, w1


def _scalar_base(lo, hi, limit):
    pc0 = (0.0 + 0.5) / float(SIDE)
    p_img = pc0 * (hi - lo) + lo
    ix = (p_img * 0.5 - 1.0) * 0.5
    return jnp.clip(ix.astype(jnp.int32), 0, limit - PW)


def _body(table_hbm, boxes_hbm, out_hbm,
          boxes_v, sa0, sw0, sa1, sw1,
          patch0_v, patch1_v, ob0_v, ob1_v,
          ps0, ps1, os0, os1):
    cid = lax.axis_index("c")
    sid = lax.axis_index("s")
    wid = sid * 2 + cid
    r0 = wid * BPW

    pltpu.sync_copy(boxes_hbm.at[pl.ds(r0 * 4, BPW * 4)],
                    boxes_v.at[pl.ds(0, BPW * 4)])

    ch_iota = lax.iota(jnp.int32, 16)
    strides = [ch_iota * P + k * 16 * P for k in range(6)]
    idx4 = ch_iota * 4

    def fire(b, patch_ref, psem, sa, sw):
        @pl.when(b < BPW)
        def _():
            bv = boxes_v[pl.ds(b * 4, 16)]
            xi0, xi1, wx0, wx1 = _tap_data(bv[0], bv[2], W)
            yi0, yi1, wy0, wy1 = _tap_data(bv[1], bv[3], H)
            bx = jnp.clip(xi0[0], 0, W - PW)
            by = jnp.clip(yi0[0], 0, H - PW)
            c0v = jnp.clip(xi0 - bx, 0, PW - 1) * C
            c1v = jnp.clip(xi1 - bx, 0, PW - 1) * C
            r0v = jnp.clip(yi0 - by, 0, PW - 1) * ROW_WORDS
            r1v = jnp.clip(yi1 - by, 0, PW - 1) * ROW_WORDS
            for l in range(SIDE):
                sa[l] = c0v[l]
                sa[16 + l] = c1v[l]
                sa[32 + l] = r0v[l]
                sa[48 + l] = r1v[l]
                sw[l] = wx0[l]
                sw[16 + l] = wx1[l]
                sw[32 + l] = wy0[l]
                sw[48 + l] = wy1[l]
            base_row = (by * W + bx) * C
            for k in range(PW):
                pltpu.async_copy(
                    table_hbm.at[pl.ds(base_row + k * (W * C), ROW_WORDS)],
                    patch_ref.at[pl.ds(k * ROW_WORDS, ROW_WORDS)],
                    psem,
                )

    def work(b, patch_ref, psem, sa, sw, ob_ref, osem):
        pltpu.make_async_copy(
            table_hbm.at[pl.ds(0, PATCH_WORDS)], patch_ref, psem).wait()

        @pl.when(b >= 2)
        def _():
            pltpu.make_async_copy(
                out_hbm.at[pl.ds(0, OBLK)], ob_ref, osem).wait()

        @plsc.parallel_loop(0, SIDE, unroll=2)
        def i_body(i):
            row0 = sa[32 + i]
            row1 = sa[48 + i]
            wy0i = sw[32 + i]
            wy1i = sw[48 + i]

            @plsc.parallel_loop(0, SIDE, unroll=14)
            def j_body(j):
                c0 = sa[j]
                c1 = sa[16 + j]
                wx0j = sw[j]
                wx1j = sw[16 + j]
                w00 = wy0i * wx0j
                w01 = wy0i * wx1j
                w10 = wy1i * wx0j
                w11 = wy1i * wx1j
                a00 = row0 + c0
                a01 = row0 + c1
                a10 = row1 + c0
                a11 = row1 + c1
                p = i * SIDE + j
                for k in range(6):
                    o = k * 16
                    v = (patch_ref[pl.ds(a00 + o, 16)] * w00
                         + patch_ref[pl.ds(a01 + o, 16)] * w01
                         + patch_ref[pl.ds(a10 + o, 16)] * w10
                         + patch_ref[pl.ds(a11 + o, 16)] * w11)
                    plsc.store_scatter(ob_ref, [strides[k] + p], v)


        pltpu.async_copy(ob_ref, out_hbm.at[pl.ds((r0 + b) * OBLK, OBLK)],
                         osem)

    fire(0, patch0_v, ps0, sa0, sw0)
    fire(1, patch1_v, ps1, sa1, sw1)

    def g_body(g, _):
        b0 = g * 2
        work(b0, patch0_v, ps0, sa0, sw0, ob0_v, os0)
        fire(b0 + 2, patch0_v, ps0, sa0, sw0)
        work(b0 + 1, patch1_v, ps1, sa1, sw1, ob1_v, os1)
        fire(b0 + 3, patch1_v, ps1, sa1, sw1)
        return ()

    lax.fori_loop(0, BPW // 2, g_body, (), unroll=False)

    pltpu.make_async_copy(out_hbm.at[pl.ds(0, OBLK)], ob0_v, os0).wait()
    pltpu.make_async_copy(out_hbm.at[pl.ds(0, OBLK)], ob1_v, os1).wait()


@jax.jit
def kernel(feature0, pred_boxes):
    table = jnp.transpose(feature0[0].reshape(C, H * W)).reshape(-1)
    boxes = jnp.pad(pred_boxes, ((0, NBP - NB), (0, 0))).reshape(-1)

    run = pl.kernel(
        _body,
        out_type=jax.ShapeDtypeStruct((NBP * OBLK,), jnp.float32),
        mesh=plsc.VectorSubcoreMesh(core_axis_name="c", subcore_axis_name="s"),
        compiler_params=pltpu.CompilerParams(needs_layout_passes=False),
        scratch_types=[
            pltpu.VMEM((BPW * 4 + 24,), jnp.float32),
            pltpu.SMEM((64,), jnp.int32),
            pltpu.SMEM((64,), jnp.float32),
            pltpu.SMEM((64,), jnp.int32),
            pltpu.SMEM((64,), jnp.float32),
            pltpu.VMEM((PATCH_WORDS,), jnp.float32),
            pltpu.VMEM((PATCH_WORDS,), jnp.float32),
            pltpu.VMEM((OBLK,), jnp.float32),
            pltpu.VMEM((OBLK,), jnp.float32),
            pltpu.SemaphoreType.DMA,
            pltpu.SemaphoreType.DMA,
            pltpu.SemaphoreType.DMA,
            pltpu.SemaphoreType.DMA,
        ],
    )
    out = run(table, boxes)
    return out.reshape(NBP, C, P)[:NB]


# speedup vs baseline: 3.3582x; 1.0101x over previous
import jax
import jax.numpy as jnp
from jax import lax
from jax.experimental import pallas as pl
from jax.experimental.pallas import tpu as pltpu
from jax.experimental.pallas import tpu_sc as plsc

C = 96
H = 128
W = 128
SIDE = 14
P = SIDE * SIDE
NB = 1000
NBP = 1024
NW = 32
BPW = NBP // NW
PW = 17
PATCH_WORDS = PW * PW * C
OBLK = C * P
ROW_WORDS = PW * C


def _tap_data(coord_lo, coord_hi, limit):
    jf = lax.iota(jnp.int32, 16).astype(jnp.float32)
    pc = (jf + 0.5) / float(SIDE)
    p_img = pc * (coord_hi - coord_lo) + coord_lo
    ix = (p_img * 0.5 - 1.0) * 0.5
    t = ix.astype(jnp.int32)
    tf = t.astype(jnp.float32)
    ione = jnp.full((16,), 1, jnp.int32)
    izero = jnp.full((16,), 0, jnp.int32)
    fone = jnp.full((16,), 1.0, jnp.float32)
    fzero = jnp.full((16,), 0.0, jnp.float32)
    i0 = t - jnp.where(tf > ix, ione, izero)
    i0f = i0.astype(jnp.float32)
    w1 = ix - i0f
    w0 = 1.0 - w1
    w0 = w0 * jnp.where(i0 >= 0, fone, fzero)
    w1 = w1 * jnp.where(i0 <= limit - 2, fone, fzero)
    return i0, i0 + 1, w0, w1


def _scalar_base(lo, hi, limit):
    pc0 = (0.0 + 0.5) / float(SIDE)
    p_img = pc0 * (hi - lo) + lo
    ix = (p_img * 0.5 - 1.0) * 0.5
    return jnp.clip(ix.astype(jnp.int32), 0, limit - PW)


def _body(table_hbm, boxes_hbm, out_hbm,
          boxes_v, jdat0_v, idat0_v, jdat1_v, idat1_v,
          patch0_v, patch1_v, ob0_v, ob1_v,
          ps0, ps1, os0, os1):
    cid = lax.axis_index("c")
    sid = lax.axis_index("s")
    wid = sid * 2 + cid
    r0 = wid * BPW

    pltpu.sync_copy(boxes_hbm.at[pl.ds(r0 * 4, BPW * 4)],
                    boxes_v.at[pl.ds(0, BPW * 4)])

    ch_iota = lax.iota(jnp.int32, 16)
    strides = [ch_iota * P + k * 16 * P for k in range(6)]
    idx4 = ch_iota * 4

    def fire(b, patch_ref, psem, jd, idt):
        @pl.when(b < BPW)
        def _():
            bv = boxes_v[pl.ds(b * 4, 16)]
            xi0, xi1, wx0, wx1 = _tap_data(bv[0], bv[2], W)
            yi0, yi1, wy0, wy1 = _tap_data(bv[1], bv[3], H)
            bx = jnp.clip(xi0[0], 0, W - PW)
            by = jnp.clip(yi0[0], 0, H - PW)
            cf = jnp.full((16,), float(C), jnp.float32)
            rf = jnp.full((16,), float(ROW_WORDS), jnp.float32)
            plsc.store_scatter(
                jd, [idx4 + 0],
                jnp.clip(xi0 - bx, 0, PW - 1).astype(jnp.float32) * cf)
            plsc.store_scatter(
                jd, [idx4 + 1],
                jnp.clip(xi1 - bx, 0, PW - 1).astype(jnp.float32) * cf)
            plsc.store_scatter(jd, [idx4 + 2], wx0)
            plsc.store_scatter(jd, [idx4 + 3], wx1)
            plsc.store_scatter(
                idt, [idx4 + 0],
                jnp.clip(yi0 - by, 0, PW - 1).astype(jnp.float32) * rf)
            plsc.store_scatter(
                idt, [idx4 + 1],
                jnp.clip(yi1 - by, 0, PW - 1).astype(jnp.float32) * rf)
            plsc.store_scatter(idt, [idx4 + 2], wy0)
            plsc.store_scatter(idt, [idx4 + 3], wy1)
            base_row = (by * W + bx) * C
            for k in range(PW):
                pltpu.async_copy(
                    table_hbm.at[pl.ds(base_row + k * (W * C), ROW_WORDS)],
                    patch_ref.at[pl.ds(k * ROW_WORDS, ROW_WORDS)],
                    psem,
                )

    def work(b, patch_ref, psem, jd, idt, ob_ref, osem):
        pltpu.make_async_copy(
            table_hbm.at[pl.ds(0, PATCH_WORDS)], patch_ref, psem).wait()

        @pl.when(b >= 2)
        def _():
            pltpu.make_async_copy(
                out_hbm.at[pl.ds(0, OBLK)], ob_ref, osem).wait()

        @plsc.parallel_loop(0, SIDE, unroll=2)
        def i_body(i):
            idv = idt[pl.ds(i * 4, 16)]
            row0 = idv[0].astype(jnp.int32)
            row1 = idv[1].astype(jnp.int32)
            wy0i = idv[2]
            wy1i = idv[3]

            @plsc.parallel_loop(0, SIDE, unroll=14)
            def j_body(j):
                jdv = jd[pl.ds(j * 4, 16)]
                c0 = jdv[0].astype(jnp.int32)
                c1 = jdv[1].astype(jnp.int32)
                wx0j = jdv[2]
                wx1j = jdv[3]
                w00 = wy0i * wx0j
                w01 = wy0i * wx1j
                w10 = wy1i * wx0j
                w11 = wy1i * wx1j
                a00 = row0 + c0
                a01 = row0 + c1
                a10 = row1 + c0
                a11 = row1 + c1
                p = i * SIDE + j
                for k in range(6):
                    o = k * 16
                    v = (patch_ref[pl.ds(a00 + o, 16)] * w00
                         + patch_ref[pl.ds(a01 + o, 16)] * w01
                         + patch_ref[pl.ds(a10 + o, 16)] * w10
                         + patch_ref[pl.ds(a11 + o, 16)] * w11)
                    plsc.store_scatter(ob_ref, [strides[k] + p], v)


        pltpu.async_copy(ob_ref, out_hbm.at[pl.ds((r0 + b) * OBLK, OBLK)],
                         osem)

    fire(0, patch0_v, ps0, jdat0_v, idat0_v)
    fire(1, patch1_v, ps1, jdat1_v, idat1_v)

    def g_body(g, _):
        b0 = g * 2
        work(b0, patch0_v, ps0, jdat0_v, idat0_v, ob0_v, os0)
        fire(b0 + 2, patch0_v, ps0, jdat0_v, idat0_v)
        work(b0 + 1, patch1_v, ps1, jdat1_v, idat1_v, ob1_v, os1)
        fire(b0 + 3, patch1_v, ps1, jdat1_v, idat1_v)
        return ()

    lax.fori_loop(0, BPW // 2, g_body, (), unroll=False)

    pltpu.make_async_copy(out_hbm.at[pl.ds(0, OBLK)], ob0_v, os0).wait()
    pltpu.make_async_copy(out_hbm.at[pl.ds(0, OBLK)], ob1_v, os1).wait()


@jax.jit
def kernel(feature0, pred_boxes):
    table = jnp.transpose(feature0[0].reshape(C, H * W)).reshape(-1)
    boxes = jnp.pad(pred_boxes, ((0, NBP - NB), (0, 0))).reshape(-1)

    run = pl.kernel(
        _body,
        out_type=jax.ShapeDtypeStruct((NBP * OBLK,), jnp.float32),
        mesh=plsc.VectorSubcoreMesh(core_axis_name="c", subcore_axis_name="s"),
        compiler_params=pltpu.CompilerParams(needs_layout_passes=False),
        scratch_types=[
            pltpu.VMEM((BPW * 4 + 24,), jnp.float32),
            pltpu.VMEM((80,), jnp.float32),
            pltpu.VMEM((80,), jnp.float32),
            pltpu.VMEM((80,), jnp.float32),
            pltpu.VMEM((80,), jnp.float32),
            pltpu.VMEM((PATCH_WORDS,), jnp.float32),
            pltpu.VMEM((PATCH_WORDS,), jnp.float32),
            pltpu.VMEM((OBLK,), jnp.float32),
            pltpu.VMEM((OBLK,), jnp.float32),
            pltpu.SemaphoreType.DMA,
            pltpu.SemaphoreType.DMA,
            pltpu.SemaphoreType.DMA,
            pltpu.SemaphoreType.DMA,
        ],
    )
    out = run(table, boxes)
    return out.reshape(NBP, C, P)[:NB]


# speedup vs baseline: 3.3640x; 1.0017x over previous
import jax
import jax.numpy as jnp
from jax import lax
from jax.experimental import pallas as pl
from jax.experimental.pallas import tpu as pltpu
from jax.experimental.pallas import tpu_sc as plsc

C = 96
H = 128
W = 128
SIDE = 14
P = SIDE * SIDE
NB = 1000
NBP = 1024
NW = 32
BPW = NBP // NW
PW = 17
PATCH_WORDS = PW * PW * C
OBLK = C * P
ROW_WORDS = PW * C


def _tap_data(coord_lo, coord_hi, limit):
    jf = lax.iota(jnp.int32, 16).astype(jnp.float32)
    pc = (jf + 0.5) / float(SIDE)
    p_img = pc * (coord_hi - coord_lo) + coord_lo
    ix = (p_img * 0.5 - 1.0) * 0.5
    t = ix.astype(jnp.int32)
    tf = t.astype(jnp.float32)
    ione = jnp.full((16,), 1, jnp.int32)
    izero = jnp.full((16,), 0, jnp.int32)
    fone = jnp.full((16,), 1.0, jnp.float32)
    fzero = jnp.full((16,), 0.0, jnp.float32)
    i0 = t - jnp.where(tf > ix, ione, izero)
    i0f = i0.astype(jnp.float32)
    w1 = ix - i0f
    w0 = 1.0 - w1
    w0 = w0 * jnp.where(i0 >= 0, fone, fzero)
    w1 = w1 * jnp.where(i0 <= limit - 2, fone, fzero)
    return i0, i0 + 1, w0, w1


def _scalar_base(lo, hi, limit):
    pc0 = (0.0 + 0.5) / float(SIDE)
    p_img = pc0 * (hi - lo) + lo
    ix = (p_img * 0.5 - 1.0) * 0.5
    return jnp.clip(ix.astype(jnp.int32), 0, limit - PW)


def _body(table_hbm, boxes_hbm, out_hbm,
          boxes_v, sa0, sw0, sa1, sw1,
          patch0_v, patch1_v, ob0_v, ob1_v,
          ps0, ps1, os0, os1):
    cid = lax.axis_index("c")
    sid = lax.axis_index("s")
    wid = sid * 2 + cid
    r0 = wid * BPW

    pltpu.sync_copy(boxes_hbm.at[pl.ds(r0 * 4, BPW * 4)],
                    boxes_v.at[pl.ds(0, BPW * 4)])

    ch_iota = lax.iota(jnp.int32, 16)
    strides = [ch_iota * P + k * 16 * P for k in range(6)]
    idx4 = ch_iota * 4

    def fire(b, patch_ref, psem, sa, sw):
        @pl.when(b < BPW)
        def _():
            bv = boxes_v[pl.ds(b * 4, 16)]
            xi0, xi1, wx0, wx1 = _tap_data(bv[0], bv[2], W)
            yi0, yi1, wy0, wy1 = _tap_data(bv[1], bv[3], H)
            bx = jnp.clip(xi0[0], 0, W - PW)
            by = jnp.clip(yi0[0], 0, H - PW)
            c0v = jnp.clip(xi0 - bx, 0, PW - 1) * C
            c1v = jnp.clip(xi1 - bx, 0, PW - 1) * C
            r0v = jnp.clip(yi0 - by, 0, PW - 1) * ROW_WORDS
            r1v = jnp.clip(yi1 - by, 0, PW - 1) * ROW_WORDS
            for l in range(SIDE):
                sa[l] = c0v[l]
                sa[16 + l] = c1v[l]
                sa[32 + l] = r0v[l]
                sa[48 + l] = r1v[l]
                sw[l] = wx0[l]
                sw[16 + l] = wx1[l]
                sw[32 + l] = wy0[l]
                sw[48 + l] = wy1[l]
            base_row = (by * W + bx) * C
            for k in range(PW):
                pltpu.async_copy(
                    table_hbm.at[pl.ds(base_row + k * (W * C), ROW_WORDS)],
                    patch_ref.at[pl.ds(k * ROW_WORDS, ROW_WORDS)],
                    psem,
                )

    def work(b, patch_ref, psem, sa, sw, ob_ref, osem):
        pltpu.make_async_copy(
            table_hbm.at[pl.ds(0, PATCH_WORDS)], patch_ref, psem).wait()

        @pl.when(b >= 2)
        def _():
            pltpu.make_async_copy(
                out_hbm.at[pl.ds(0, OBLK)], ob_ref, osem).wait()

        @plsc.parallel_loop(0, P, unroll=14)
        def p_body(p):
            i = p // SIDE
            j = p - i * SIDE
            row0 = sa[32 + i]
            row1 = sa[48 + i]
            wy0i = sw[32 + i]
            wy1i = sw[48 + i]
            if True:
                c0 = sa[j]
                c1 = sa[16 + j]
                wx0j = sw[j]
                wx1j = sw[16 + j]
                w00 = wy0i * wx0j
                w01 = wy0i * wx1j
                w10 = wy1i * wx0j
                w11 = wy1i * wx1j
                a00 = row0 + c0
                a01 = row0 + c1
                a10 = row1 + c0
                a11 = row1 + c1
                for k in range(6):
                    o = k * 16
                    v = (patch_ref[pl.ds(a00 + o, 16)] * w00
                         + patch_ref[pl.ds(a01 + o, 16)] * w01
                         + patch_ref[pl.ds(a10 + o, 16)] * w10
                         + patch_ref[pl.ds(a11 + o, 16)] * w11)
                    plsc.store_scatter(ob_ref, [strides[k] + p], v)


        pltpu.async_copy(ob_ref, out_hbm.at[pl.ds((r0 + b) * OBLK, OBLK)],
                         osem)

    fire(0, patch0_v, ps0, sa0, sw0)
    fire(1, patch1_v, ps1, sa1, sw1)

    def g_body(g, _):
        b0 = g * 2
        work(b0, patch0_v, ps0, sa0, sw0, ob0_v, os0)
        fire(b0 + 2, patch0_v, ps0, sa0, sw0)
        work(b0 + 1, patch1_v, ps1, sa1, sw1, ob1_v, os1)
        fire(b0 + 3, patch1_v, ps1, sa1, sw1)
        return ()

    lax.fori_loop(0, BPW // 2, g_body, (), unroll=False)

    pltpu.make_async_copy(out_hbm.at[pl.ds(0, OBLK)], ob0_v, os0).wait()
    pltpu.make_async_copy(out_hbm.at[pl.ds(0, OBLK)], ob1_v, os1).wait()


@jax.jit
def kernel(feature0, pred_boxes):
    table = jnp.transpose(feature0[0].reshape(C, H * W)).reshape(-1)
    boxes = jnp.pad(pred_boxes, ((0, NBP - NB), (0, 0))).reshape(-1)

    run = pl.kernel(
        _body,
        out_type=jax.ShapeDtypeStruct((NBP * OBLK,), jnp.float32),
        mesh=plsc.VectorSubcoreMesh(core_axis_name="c", subcore_axis_name="s"),
        compiler_params=pltpu.CompilerParams(needs_layout_passes=False),
        scratch_types=[
            pltpu.VMEM((BPW * 4 + 24,), jnp.float32),
            pltpu.SMEM((64,), jnp.int32),
            pltpu.SMEM((64,), jnp.float32),
            pltpu.SMEM((64,), jnp.int32),
            pltpu.SMEM((64,), jnp.float32),
            pltpu.VMEM((PATCH_WORDS,), jnp.float32),
            pltpu.VMEM((PATCH_WORDS,), jnp.float32),
            pltpu.VMEM((OBLK,), jnp.float32),
            pltpu.VMEM((OBLK,), jnp.float32),
            pltpu.SemaphoreType.DMA,
            pltpu.SemaphoreType.DMA,
            pltpu.SemaphoreType.DMA,
            pltpu.SemaphoreType.DMA,
        ],
    )
    out = run(table, boxes)
    return out.reshape(NBP, C, P)[:NB]
